# baseline (speedup 1.0000x reference)
import functools

import jax
import jax.numpy as jnp
from jax import lax
from jax.experimental import pallas as pl
from jax.experimental.pallas import tpu as pltpu
from jax.experimental.pallas import tpu_sc as plsc

BN_EPS = 1e-5
_CHUNK = 256


def _sc_dual_gather(user_table, movie_table, user_idx, movie_idx):
    B = user_idx.shape[0]
    E = user_table.shape[1]
    info = plsc.get_sparse_core_info()
    NC, NS = info.num_cores, info.num_subcores
    NW = NC * NS
    b_per_w = B // NW
    mesh = plsc.VectorSubcoreMesh(core_axis_name="c", subcore_axis_name="s")
    u3 = user_table.reshape(-1, 8, E)
    m2 = movie_table

    @functools.partial(
        pl.kernel,
        mesh=mesh,
        out_type=[
            jax.ShapeDtypeStruct((B, E), jnp.float32),
            jax.ShapeDtypeStruct((B, E), jnp.float32),
        ],
        scratch_types=[
            pltpu.VMEM((b_per_w,), jnp.int32),
            pltpu.VMEM((b_per_w,), jnp.int32),
            pltpu.VMEM((_CHUNK, E), jnp.float32),
            pltpu.VMEM((_CHUNK, E), jnp.float32),
            pltpu.SemaphoreType.DMA,
            pltpu.SemaphoreType.DMA,
        ],
    )
    def gather2(u_tab, m_tab, u_idx, m_idx, u_out, m_out,
                uidx_v, midx_v, ubuf, mbuf, usem, msem):
        wid = lax.axis_index("s") * NC + lax.axis_index("c")
        base = wid * b_per_w
        pltpu.sync_copy(u_idx.at[pl.ds(base, b_per_w)], uidx_v)
        pltpu.sync_copy(m_idx.at[pl.ds(base, b_per_w)], midx_v)

        def fire(tab, idx_v, c, buf, sem, three_d):
            def body(j, carry):
                vec = idx_v[pl.ds(c * _CHUNK + j * 16, 16)]
                sup = lax.shift_right_logical(vec, 3)
                sub = lax.bitwise_and(vec, 7)
                for l in range(16):
                    if three_d:
                        src = tab.at[sup[l], pl.ds(sub[l], 1)]
                    else:
                        src = tab.at[pl.ds(vec[l], 1)]
                    pltpu.async_copy(src, buf.at[pl.ds(j * 16 + l, 1)], sem)
                return carry
            lax.fori_loop(0, _CHUNK // 16, body, 0)

        def drain(out_ref, buf, sem):
            pltpu.make_async_copy(out_ref.at[pl.ds(0, _CHUNK)], buf,
                                  sem).wait()

        def copyout(out_ref, c, buf):
            pltpu.sync_copy(buf, out_ref.at[pl.ds(base + c * _CHUNK, _CHUNK)])

        n_rounds = b_per_w // _CHUNK
        fire(u_tab, uidx_v, 0, ubuf, usem, True)
        fire(m_tab, midx_v, 0, mbuf, msem, False)
        for r in range(n_rounds):
            drain(u_out, ubuf, usem)
            copyout(u_out, r, ubuf)
            if r + 1 < n_rounds:
                fire(u_tab, uidx_v, r + 1, ubuf, usem, True)
            drain(m_out, mbuf, msem)
            copyout(m_out, r, mbuf)
            if r + 1 < n_rounds:
                fire(m_tab, midx_v, r + 1, mbuf, msem, False)

    return gather2(u3, m2, user_idx, movie_idx)


def _mlp_body(u_ref, m_ref, cf_ref, wc_ref, bc_ref,
              w1u_ref, w1m_ref, w1c_ref, b1_ref, g1_ref, be1_ref,
              w2_ref, b2_ref, g2_ref, be2_ref, w3_ref, b3_ref, out_ref):
    inv_std = 1.0 / jnp.sqrt(1.0 + BN_EPS)
    c = jnp.dot(cf_ref[...], wc_ref[...], preferred_element_type=jnp.float32)
    c = c + bc_ref[...]
    h = (jnp.dot(u_ref[...], w1u_ref[...], preferred_element_type=jnp.float32)
         + jnp.dot(m_ref[...], w1m_ref[...], preferred_element_type=jnp.float32)
         + jnp.dot(c, w1c_ref[...], preferred_element_type=jnp.float32)
         + b1_ref[...])
    h = h * (inv_std * g1_ref[...]) + be1_ref[...]
    h = jnp.maximum(h, 0.0)
    h = jnp.dot(h, w2_ref[...], preferred_element_type=jnp.float32) + b2_ref[...]
    h = h * (inv_std * g2_ref[...]) + be2_ref[...]
    h = jnp.maximum(h, 0.0)
    z = jnp.dot(h, w3_ref[...], preferred_element_type=jnp.float32) + b3_ref[...]
    out_ref[...] = 5.0 / (1.0 + jnp.exp(-z))


def _mlp(user_emb, movie_emb, content_features, Wc, bc,
         W1, b1, g1, be1, W2, b2, g2, be2, W3, b3):
    B, E = user_emb.shape
    bm = 4096
    grid = (B // bm,)
    W1u = W1[:E]
    W1m = W1[E:2 * E]
    W1c = W1[2 * E:]
    row = lambda v: v.reshape(1, -1)
    data_spec = lambda cols: pl.BlockSpec((bm, cols), lambda i: (i, 0))
    full = lambda a: pl.BlockSpec(a.shape, lambda i: (0, 0))
    out = pl.pallas_call(
        _mlp_body,
        grid=grid,
        in_specs=[
            data_spec(E), data_spec(E),
            data_spec(content_features.shape[1]),
            full(Wc), full(row(bc)),
            full(W1u), full(W1m), full(W1c),
            full(row(b1)), full(row(g1)), full(row(be1)),
            full(W2), full(row(b2)), full(row(g2)), full(row(be2)),
            full(W3), full(row(b3)),
        ],
        out_specs=pl.BlockSpec((bm, 1), lambda i: (i, 0)),
        out_shape=jax.ShapeDtypeStruct((B, 1), jnp.float32),
    )(user_emb, movie_emb, content_features,
      Wc, row(bc), W1u, W1m, W1c, row(b1), row(g1), row(be1),
      W2, row(b2), row(g2), row(be2), W3, row(b3))
    return out


def kernel(user_idx, movie_idx, content_features, user_table, movie_table,
           Wc, bc, W1, b1, g1, be1, W2, b2, g2, be2, W3, b3):
    user_emb, movie_emb = _sc_dual_gather(user_table, movie_table,
                                          user_idx, movie_idx)
    return _mlp(user_emb, movie_emb, content_features, Wc, bc,
                W1, b1, g1, be1, W2, b2, g2, be2, W3, b3)


# speedup vs baseline: 1.0839x; 1.0839x over previous
import functools

import jax
import jax.numpy as jnp
from jax import lax
from jax.experimental import pallas as pl
from jax.experimental.pallas import tpu as pltpu
from jax.experimental.pallas import tpu_sc as plsc

BN_EPS = 1e-5
_CHUNK = 256


def _sc_dual_gather(user_table, movie_table, user_idx, movie_idx):
    B = user_idx.shape[0]
    E = user_table.shape[1]
    info = plsc.get_sparse_core_info()
    NC, NS = info.num_cores, info.num_subcores
    NW = NC * NS
    b_per_w = B // NW
    mesh = plsc.VectorSubcoreMesh(core_axis_name="c", subcore_axis_name="s")
    u3 = user_table.reshape(-1, 8, E)
    m3 = movie_table.reshape(-1, 8, E)

    @functools.partial(
        pl.kernel,
        mesh=mesh,
        out_type=[
            jax.ShapeDtypeStruct((B, E), jnp.float32),
            jax.ShapeDtypeStruct((B, E), jnp.float32),
        ],
        scratch_types=[
            pltpu.VMEM((b_per_w,), jnp.int32),
            pltpu.VMEM((b_per_w,), jnp.int32),
            pltpu.VMEM((_CHUNK, E), jnp.float32),
            pltpu.VMEM((_CHUNK, E), jnp.float32),
            pltpu.SemaphoreType.DMA,
            pltpu.SemaphoreType.DMA,
        ],
    )
    def gather2(u_tab, m_tab, u_idx, m_idx, u_out, m_out,
                uidx_v, midx_v, ubuf, mbuf, usem, msem):
        wid = lax.axis_index("s") * NC + lax.axis_index("c")
        base = wid * b_per_w
        pltpu.sync_copy(u_idx.at[pl.ds(base, b_per_w)], uidx_v)
        pltpu.sync_copy(m_idx.at[pl.ds(base, b_per_w)], midx_v)

        def fire(tab, idx_v, c, buf, sem):
            def body(j, carry):
                vec = idx_v[pl.ds(c * _CHUNK + j * 16, 16)]
                sup = lax.shift_right_logical(vec, 3)
                sub = lax.bitwise_and(vec, 7)
                for l in range(16):
                    pltpu.async_copy(tab.at[sup[l], pl.ds(sub[l], 1)],
                                     buf.at[pl.ds(j * 16 + l, 1)], sem)
                return carry
            lax.fori_loop(0, _CHUNK // 16, body, 0)

        def drain(out_ref, buf, sem):
            pltpu.make_async_copy(out_ref.at[pl.ds(0, _CHUNK)], buf,
                                  sem).wait()

        def copyout(out_ref, c, buf):
            pltpu.sync_copy(buf, out_ref.at[pl.ds(base + c * _CHUNK, _CHUNK)])

        n_rounds = b_per_w // _CHUNK
        fire(u_tab, uidx_v, 0, ubuf, usem)
        fire(m_tab, midx_v, 0, mbuf, msem)
        for r in range(n_rounds):
            drain(u_out, ubuf, usem)
            copyout(u_out, r, ubuf)
            if r + 1 < n_rounds:
                fire(u_tab, uidx_v, r + 1, ubuf, usem)
            drain(m_out, mbuf, msem)
            copyout(m_out, r, mbuf)
            if r + 1 < n_rounds:
                fire(m_tab, midx_v, r + 1, mbuf, msem)

    return gather2(u3, m3, user_idx, movie_idx)


def _mlp_body(u_ref, m_ref, cf_ref, wc_ref, bc_ref,
              w1u_ref, w1m_ref, w1c_ref, b1_ref, g1_ref, be1_ref,
              w2_ref, b2_ref, g2_ref, be2_ref, w3_ref, b3_ref, out_ref):
    inv_std = 1.0 / jnp.sqrt(1.0 + BN_EPS)
    c = jnp.dot(cf_ref[...], wc_ref[...], preferred_element_type=jnp.float32)
    c = c + bc_ref[...]
    h = (jnp.dot(u_ref[...], w1u_ref[...], preferred_element_type=jnp.float32)
         + jnp.dot(m_ref[...], w1m_ref[...], preferred_element_type=jnp.float32)
         + jnp.dot(c, w1c_ref[...], preferred_element_type=jnp.float32)
         + b1_ref[...])
    h = h * (inv_std * g1_ref[...]) + be1_ref[...]
    h = jnp.maximum(h, 0.0)
    h = jnp.dot(h, w2_ref[...], preferred_element_type=jnp.float32) + b2_ref[...]
    h = h * (inv_std * g2_ref[...]) + be2_ref[...]
    h = jnp.maximum(h, 0.0)
    z = jnp.dot(h, w3_ref[...], preferred_element_type=jnp.float32) + b3_ref[...]
    out_ref[...] = 5.0 / (1.0 + jnp.exp(-z))


def _mlp(user_emb, movie_emb, content_features, Wc, bc,
         W1, b1, g1, be1, W2, b2, g2, be2, W3, b3):
    B, E = user_emb.shape
    bm = 2048
    grid = (B // bm,)
    W1u = W1[:E]
    W1m = W1[E:2 * E]
    W1c = W1[2 * E:]
    row = lambda v: v.reshape(1, -1)
    data_spec = lambda cols: pl.BlockSpec((bm, cols), lambda i: (i, 0))
    full = lambda a: pl.BlockSpec(a.shape, lambda i: (0, 0))
    out = pl.pallas_call(
        _mlp_body,
        grid=grid,
        in_specs=[
            data_spec(E), data_spec(E),
            data_spec(content_features.shape[1]),
            full(Wc), full(row(bc)),
            full(W1u), full(W1m), full(W1c),
            full(row(b1)), full(row(g1)), full(row(be1)),
            full(W2), full(row(b2)), full(row(g2)), full(row(be2)),
            full(W3), full(row(b3)),
        ],
        out_specs=pl.BlockSpec((bm, 1), lambda i: (i, 0)),
        out_shape=jax.ShapeDtypeStruct((B, 1), jnp.float32),
    )(user_emb, movie_emb, content_features,
      Wc, row(bc), W1u, W1m, W1c, row(b1), row(g1), row(be1),
      W2, row(b2), row(g2), row(be2), W3, row(b3))
    return out


def kernel(user_idx, movie_idx, content_features, user_table, movie_table,
           Wc, bc, W1, b1, g1, be1, W2, b2, g2, be2, W3, b3):
    user_emb, movie_emb = _sc_dual_gather(user_table, movie_table,
                                          user_idx, movie_idx)
    return _mlp(user_emb, movie_emb, content_features, Wc, bc,
                W1, b1, g1, be1, W2, b2, g2, be2, W3, b3)
